# baseline (speedup 1.0000x reference)
import jax
import jax.numpy as jnp
from jax.experimental import pallas as pl
from jax.experimental.pallas import tpu as pltpu

_B = 16384
_HIDDEN = 128
_NUM_CHARGES = 11
_M = 8


def _fused_kernel(mz_ref, charge_ref, table_ref, w1_ref, b1_ref, w2_ref,
                  b2_ref, out_hbm, buf, sems):
    mz = mz_ref[:]
    h = jnp.maximum(mz[:, None] * w1_ref[0][None, :] + b1_ref[:][None, :], 0.0)
    emb0 = jax.lax.dot_general(
        h, w2_ref[:],
        dimension_numbers=(((1,), (0,)), ((), ())),
        preferred_element_type=jnp.float32,
    ) + b2_ref[:][None, :]

    charge = charge_ref[:]
    classes = jax.lax.broadcasted_iota(jnp.int32, (charge.shape[0],
                                                   _NUM_CHARGES), 1)
    onehot = (charge[:, None] == classes).astype(jnp.float32)
    emb1 = jax.lax.dot_general(
        onehot, table_ref[:],
        dimension_numbers=(((1,), (0,)), ((), ())),
        preferred_element_type=jnp.float32,
    )

    buf[:, :_HIDDEN] = emb0
    buf[:, _HIDDEN:] = emb1

    ch = _B // _M
    copies = [
        pltpu.make_async_copy(
            buf.at[pl.ds(m * ch, ch), :],
            out_hbm.at[pl.ds(m * ch, ch), :],
            sems.at[m],
        )
        for m in range(_M)
    ]
    for c in copies:
        c.start()
    for c in copies:
        c.wait()


@jax.jit
def kernel(precursor_mz, charge, charge_table, W1, b1, W2, b2):
    charge = charge.astype(jnp.int32)
    out = pl.pallas_call(
        _fused_kernel,
        out_specs=pl.BlockSpec(memory_space=pl.ANY),
        out_shape=jax.ShapeDtypeStruct((_B, 2 * _HIDDEN), jnp.float32),
        scratch_shapes=[
            pltpu.VMEM((_B, 2 * _HIDDEN), jnp.float32),
            pltpu.SemaphoreType.DMA((_M,)),
        ],
    )(precursor_mz, charge, charge_table, W1, b1, W2, b2)
    return out.reshape(_B, 2, _HIDDEN)


# speedup vs baseline: 1.1343x; 1.1343x over previous
import jax
import jax.numpy as jnp
from jax.experimental import pallas as pl
from jax.experimental.pallas import tpu as pltpu

_B = 16384
_HIDDEN = 128
_NUM_CHARGES = 11
_M = 8


def _fused_kernel(mz_ref, charge_ref, table_ref, w1_ref, b1_ref, w2_ref,
                  b2_ref, out_hbm, buf, sems):
    ch = _B // _M
    copies = []
    for m in range(_M):
        rows = pl.ds(m * ch, ch)
        mz = mz_ref[rows]
        h = jnp.maximum(mz[:, None] * w1_ref[0][None, :] + b1_ref[:][None, :],
                        0.0)
        emb0 = jax.lax.dot_general(
            h, w2_ref[:],
            dimension_numbers=(((1,), (0,)), ((), ())),
            preferred_element_type=jnp.float32,
        ) + b2_ref[:][None, :]

        charge = charge_ref[rows]
        classes = jax.lax.broadcasted_iota(jnp.int32, (ch, _NUM_CHARGES), 1)
        onehot = (charge[:, None] == classes).astype(jnp.float32)
        emb1 = jax.lax.dot_general(
            onehot, table_ref[:],
            dimension_numbers=(((1,), (0,)), ((), ())),
            preferred_element_type=jnp.float32,
        )

        buf[rows, :_HIDDEN] = emb0
        buf[rows, _HIDDEN:] = emb1

        copy = pltpu.make_async_copy(
            buf.at[rows, :], out_hbm.at[rows, :], sems.at[m])
        copy.start()
        copies.append(copy)
    for c in copies:
        c.wait()


@jax.jit
def kernel(precursor_mz, charge, charge_table, W1, b1, W2, b2):
    charge = charge.astype(jnp.int32)
    out = pl.pallas_call(
        _fused_kernel,
        out_specs=pl.BlockSpec(memory_space=pl.ANY),
        out_shape=jax.ShapeDtypeStruct((_B, 2 * _HIDDEN), jnp.float32),
        scratch_shapes=[
            pltpu.VMEM((_B, 2 * _HIDDEN), jnp.float32),
            pltpu.SemaphoreType.DMA((_M,)),
        ],
    )(precursor_mz, charge, charge_table, W1, b1, W2, b2)
    return out.reshape(_B, 2, _HIDDEN)


# speedup vs baseline: 1.1403x; 1.0053x over previous
import jax
import jax.numpy as jnp
from jax.experimental import pallas as pl
from jax.experimental.pallas import tpu as pltpu

_B = 16384
_HIDDEN = 128
_NUM_CHARGES = 11
_M = 16


def _fused_kernel(mz_ref, charge_ref, table_ref, w1_ref, b1_ref, w2_ref,
                  b2_ref, out_hbm, buf, sems):
    ch = _B // _M
    copies = []
    for m in range(_M):
        rows = pl.ds(m * ch, ch)
        mz = mz_ref[rows]
        h = jnp.maximum(mz[:, None] * w1_ref[0][None, :] + b1_ref[:][None, :],
                        0.0)
        emb0 = jax.lax.dot_general(
            h, w2_ref[:],
            dimension_numbers=(((1,), (0,)), ((), ())),
            preferred_element_type=jnp.float32,
        ) + b2_ref[:][None, :]

        charge = charge_ref[rows]
        classes = jax.lax.broadcasted_iota(jnp.int32, (ch, _NUM_CHARGES), 1)
        onehot = (charge[:, None] == classes).astype(jnp.float32)
        emb1 = jax.lax.dot_general(
            onehot, table_ref[:],
            dimension_numbers=(((1,), (0,)), ((), ())),
            preferred_element_type=jnp.float32,
        )

        buf[rows, :_HIDDEN] = emb0
        buf[rows, _HIDDEN:] = emb1

        copy = pltpu.make_async_copy(
            buf.at[rows, :], out_hbm.at[rows, :], sems.at[m])
        copy.start()
        copies.append(copy)
    for c in copies:
        c.wait()


@jax.jit
def kernel(precursor_mz, charge, charge_table, W1, b1, W2, b2):
    charge = charge.astype(jnp.int32)
    out = pl.pallas_call(
        _fused_kernel,
        out_specs=pl.BlockSpec(memory_space=pl.ANY),
        out_shape=jax.ShapeDtypeStruct((_B, 2 * _HIDDEN), jnp.float32),
        scratch_shapes=[
            pltpu.VMEM((_B, 2 * _HIDDEN), jnp.float32),
            pltpu.SemaphoreType.DMA((_M,)),
        ],
    )(precursor_mz, charge, charge_table, W1, b1, W2, b2)
    return out.reshape(_B, 2, _HIDDEN)
